# baseline (speedup 1.0000x reference)
import jax
import jax.numpy as jnp
from jax.experimental import pallas as pl

N = 10000
E = 320000
EMB = 128
HID = 128
REL = 64
PE = 31
L = 2

EDGE_BLK = 512


def _edge_kernel(xcg_ref, rel_ref, peg_ref, wc_ref, bc_ref, wg1_ref, bg1_ref,
                 wg2_ref, bg2_ref, m_ref):
    rc = jnp.dot(rel_ref[...], wc_ref[...], preferred_element_type=jnp.float32)
    msg = jnp.maximum(xcg_ref[...] + rc + bc_ref[...], 0.0)
    h = jnp.maximum(
        jnp.dot(peg_ref[...], wg1_ref[...], preferred_element_type=jnp.float32)
        + bg1_ref[...], 0.0)
    logit = jnp.dot(h, wg2_ref[...], preferred_element_type=jnp.float32)
    gate = jax.nn.sigmoid(logit[:, :1] + bg2_ref[0, 0])
    m_ref[...] = gate * msg


def _edge_pass(xcg, rel, peg, wc, bc, wg1, bg1, wg2, bg2):
    return pl.pallas_call(
        _edge_kernel,
        grid=(E // EDGE_BLK,),
        in_specs=[
            pl.BlockSpec((EDGE_BLK, HID), lambda i: (i, 0)),
            pl.BlockSpec((EDGE_BLK, EMB), lambda i: (i, 0)),
            pl.BlockSpec((EDGE_BLK, 64), lambda i: (i, 0)),
            pl.BlockSpec((EMB, HID), lambda i: (0, 0)),
            pl.BlockSpec((1, HID), lambda i: (0, 0)),
            pl.BlockSpec((64, 64), lambda i: (0, 0)),
            pl.BlockSpec((1, 64), lambda i: (0, 0)),
            pl.BlockSpec((64, 128), lambda i: (0, 0)),
            pl.BlockSpec((1, 1), lambda i: (0, 0)),
        ],
        out_specs=pl.BlockSpec((EDGE_BLK, HID), lambda i: (i, 0)),
        out_shape=jax.ShapeDtypeStruct((E, HID), jnp.float32),
    )(xcg, rel, peg, wc, bc, wg1, bg1, wg2, bg2)


def kernel(entity_embs, pe, edge_index, relation_embs_per_edge, W_in, b_in,
           W_rel, b_rel, W_msg, b_msg, W_g1, b_g1, W_g2, b_g2, W_upd, b_upd,
           W_out, b_out):
    src = edge_index[0]
    dst = edge_index[1]

    x = jax.nn.relu(jnp.concatenate([entity_embs, pe], axis=-1) @ W_in + b_in)

    pe_j = pe[src]
    pe_i = pe[dst]
    peg = jnp.concatenate(
        [pe_j, pe_i, jnp.zeros((E, 64 - 2 * PE), jnp.float32)], axis=-1)

    cnt = jnp.zeros((N, 1), jnp.float32).at[dst].add(1.0)
    cnt = jnp.maximum(cnt, 1.0)

    for l in range(L):
        wc = W_rel @ W_msg[l][HID:]
        bc = (b_rel @ W_msg[l][HID:] + b_msg[l]).reshape(1, HID)
        wg1 = jnp.concatenate(
            [W_g1[l], jnp.zeros((64 - 2 * PE, 64), jnp.float32)], axis=0)
        wg2 = jnp.concatenate([W_g2[l], jnp.zeros((64, 127), jnp.float32)],
                              axis=1)
        xc = x @ W_msg[l][:HID]
        xcg = xc[src]
        m = _edge_pass(xcg, relation_embs_per_edge, peg, wc, bc, wg1,
                       b_g1[l].reshape(1, 64), wg2, b_g2[l].reshape(1, 1))
        agg = jnp.zeros((N, HID), jnp.float32).at[dst].add(m) / cnt
        x = jax.nn.relu(jnp.concatenate([x, agg], axis=-1) @ W_upd[l] + b_upd[l])

    return x @ W_out + b_out


# speedup vs baseline: 3.4238x; 3.4238x over previous
import functools

import jax
import jax.numpy as jnp
from jax import lax
from jax.experimental import pallas as pl
from jax.experimental.pallas import tpu as pltpu
from jax.experimental.pallas import tpu_sc as plsc

N = 10000
E = 320000
EMB = 128
HID = 128
REL = 64
PE = 31
L = 2

BLK_E = 512
BLK_N = 1000
KE = 128
NBLK_E = E // KE
TILE_ROWS = 624
_CHUNKS = ((0, 128), (128, 128), (256, 128), (384, 128), (512, 112))
_REM_OFF = 16 * TILE_ROWS
_REM = N - _REM_OFF

_mesh = plsc.VectorSubcoreMesh(core_axis_name="c", subcore_axis_name="s")


def _proj_in_kernel(ee_ref, pe_ref, wt_ref, wb_ref, bi_ref, wmt_ref,
                    a0_ref, a1_ref, b0_ref, b1_ref, g0_ref, g1_ref,
                    x_ref, xc_ref, s_ref, d_ref):
    pe = pe_ref[...]
    x = jnp.dot(ee_ref[...], wt_ref[...], preferred_element_type=jnp.float32)
    x = x + jnp.dot(pe, wb_ref[...], preferred_element_type=jnp.float32)
    x = jnp.maximum(x + bi_ref[...], 0.0)
    x_ref[...] = x
    xc_ref[...] = jnp.dot(x, wmt_ref[...], preferred_element_type=jnp.float32)
    s0 = jnp.dot(pe, a0_ref[...], preferred_element_type=jnp.float32)
    s1 = jnp.dot(pe, a1_ref[...], preferred_element_type=jnp.float32)
    s_ref[...] = jnp.concatenate([s0, s1], axis=1)
    d0 = jnp.dot(pe, b0_ref[...], preferred_element_type=jnp.float32)
    d1 = jnp.dot(pe, b1_ref[...], preferred_element_type=jnp.float32)
    d_ref[...] = jnp.concatenate([d0 + g0_ref[...], d1 + g1_ref[...]], axis=1)


def _rc_kernel(rel_ref, wr_ref, br_ref, wm_ref, bm_ref, rc_ref):
    ea = jnp.dot(rel_ref[...], wr_ref[...], preferred_element_type=jnp.float32)
    ea = ea + br_ref[...]
    rc = jnp.dot(ea, wm_ref[...], preferred_element_type=jnp.float32)
    rc_ref[...] = rc + bm_ref[...]


def _upd_kernel(x_ref, aggp_ref, cnt_ref, wt_ref, wb_ref, b_ref, wn_ref,
                bn_ref, x2_ref, xc2_ref, final):
    aggs = aggp_ref[0] + aggp_ref[1]
    cnt = jnp.maximum(cnt_ref[0, :, :1] + cnt_ref[1, :, :1], 1.0)
    agg = aggs / cnt
    x = jnp.dot(x_ref[...], wt_ref[...], preferred_element_type=jnp.float32)
    x = x + jnp.dot(agg, wb_ref[...], preferred_element_type=jnp.float32)
    x = jnp.maximum(x + b_ref[...], 0.0)
    x2_ref[...] = x
    y = jnp.dot(x, wn_ref[...], preferred_element_type=jnp.float32)
    if final:
        xc2_ref[...] = y + bn_ref[...]
    else:
        xc2_ref[...] = y


_GDN = lax.GatherDimensionNumbers(
    offset_dims=(), collapsed_slice_dims=(0,), start_index_map=(0,))


def _allsum(v):
    for s in (8, 4, 2, 1):
        idx = lax.iota(jnp.int32, 16) ^ s
        v = v + lax.gather(v, idx[:, None], _GDN, (1,),
                           mode=lax.GatherScatterMode.PROMISE_IN_BOUNDS)
    return v


def _tile_sweep(sid, copy_fn):
    row0 = sid * TILE_ROWS
    for off, nr in _CHUNKS:
        copy_fn(row0 + off, nr)

    @pl.when(sid == 0)
    def _():
        copy_fn(_REM_OFF, _REM)


MAX_WIN = (NBLK_E + 31) // 32
KG = 64
NBLK_G = E // KG
MAX_WIN_G = (NBLK_G + 31) // 32


def _gate_from_rows(sg_v, dg_v, e, w2v, bg2v, l):
    acc = None
    for j in range(4):
        c = 64 * l + 16 * j
        h = jnp.maximum(sg_v[e, c:c + 16] + dg_v[e, c:c + 16], 0.0)
        t = h * w2v[4 * l + j]
        acc = t if acc is None else acc + t
    logit = _allsum(acc) + bg2v[l]
    return 1.0 / (1.0 + jnp.exp(-logit))


def _gate_cnt_body(src_hbm, dst_hbm, s_hbm, d_hbm, w2e_hbm,
                   gate0_hbm, gate1_hbm, cnt_hbm,
                   sidx, didx, sg_v, dg_v, g0_v, g1_v, ones_v, w2_v,
                   cnt_sh, sem):
    cid = lax.axis_index("c")
    sid = lax.axis_index("s")
    wid = sid * 2 + cid

    def fill(i, _):
        for j in range(8):
            ones_v[i, 16 * j:16 * (j + 1)] = jnp.zeros((16,), jnp.float32)
        return 0
    lax.fori_loop(0, KG, fill, 0)

    pltpu.sync_copy(w2e_hbm, w2_v)

    row0 = sid * TILE_ROWS
    for k in range(9):
        pltpu.sync_copy(ones_v, cnt_sh.at[pl.ds(row0 + 64 * k, 64)])
    pltpu.sync_copy(ones_v.at[pl.ds(0, 48)],
                    cnt_sh.at[pl.ds(row0 + 576, 48)])

    @pl.when(sid == 0)
    def _():
        pltpu.sync_copy(ones_v.at[pl.ds(0, _REM)],
                        cnt_sh.at[pl.ds(_REM_OFF, _REM)])

    def refill(i, _):
        for j in range(8):
            ones_v[i, 16 * j:16 * (j + 1)] = jnp.ones((16,), jnp.float32)
        return 0
    lax.fori_loop(0, KG, refill, 0)
    plsc.subcore_barrier()

    w2v = [w2_v[j // 4, 16 * (j % 4):16 * (j % 4) + 16] for j in range(8)]
    bg2v = [w2_v[l, 64:80] for l in range(L)]

    def body(i, _):
        b = wid + 32 * i

        @pl.when(b < NBLK_G)
        def _():
            base = b * KG
            pltpu.sync_copy(src_hbm.at[pl.ds(base, KG)], sidx)
            pltpu.sync_copy(dst_hbm.at[pl.ds(base, KG)], didx)
            cp2 = pltpu.async_copy(s_hbm.at[sidx], sg_v, sem)
            cp3 = pltpu.async_copy(d_hbm.at[didx], dg_v, sem)
            cp2.wait()
            cp3.wait()

            def ebody(e, _):
                g0_v[e, :] = _gate_from_rows(sg_v, dg_v, e, w2v, bg2v, 0)
                g1_v[e, :] = _gate_from_rows(sg_v, dg_v, e, w2v, bg2v, 1)
                return 0
            lax.fori_loop(0, KG, ebody, 0)

            pltpu.sync_copy(ones_v, cnt_sh.at[didx], add=True)
            pltpu.sync_copy(g0_v, gate0_hbm.at[pl.ds(base, KG)])
            pltpu.sync_copy(g1_v, gate1_hbm.at[pl.ds(base, KG)])
        return 0
    lax.fori_loop(0, MAX_WIN_G, body, 0)

    plsc.subcore_barrier()
    _tile_sweep(sid, lambda r, n: pltpu.sync_copy(
        cnt_sh.at[pl.ds(r, n)], cnt_hbm.at[cid, pl.ds(r, n)]))


def _agg_body(src_hbm, dst_hbm, xc_hbm, rc_hbm, gate_hbm,
              agg_hbm,
              sidx, didx, xcg_v, rc_v, gate_v, agg_sh, sem):
    cid = lax.axis_index("c")
    sid = lax.axis_index("s")
    wid = sid * 2 + cid

    def fill(i, _):
        for j in range(8):
            rc_v[i, 16 * j:16 * (j + 1)] = jnp.zeros((16,), jnp.float32)
        return 0
    lax.fori_loop(0, KE, fill, 0)

    _tile_sweep(sid, lambda r, n: pltpu.sync_copy(
        rc_v.at[pl.ds(0, n)], agg_sh.at[pl.ds(r, n)]))
    plsc.subcore_barrier()

    def body(i, _):
        b = wid + 32 * i

        @pl.when(b < NBLK_E)
        def _():
            base = b * KE
            pltpu.sync_copy(src_hbm.at[pl.ds(base, KE)], sidx)
            pltpu.sync_copy(dst_hbm.at[pl.ds(base, KE)], didx)
            cp1 = pltpu.async_copy(xc_hbm.at[sidx], xcg_v, sem)
            pltpu.sync_copy(rc_hbm.at[pl.ds(base, KE)], rc_v)
            pltpu.sync_copy(gate_hbm.at[pl.ds(base, KE)], gate_v)
            cp1.wait()

            def ebody(e, _):
                g = gate_v[e, :]
                for j in range(8):
                    sl = slice(16 * j, 16 * (j + 1))
                    rc_v[e, sl] = g * jnp.maximum(
                        xcg_v[e, sl] + rc_v[e, sl], 0.0)
                return 0
            lax.fori_loop(0, KE, ebody, 0)

            pltpu.sync_copy(rc_v, agg_sh.at[didx], add=True)
        return 0
    lax.fori_loop(0, MAX_WIN, body, 0)

    plsc.subcore_barrier()
    _tile_sweep(sid, lambda r, n: pltpu.sync_copy(
        agg_sh.at[pl.ds(r, n)], agg_hbm.at[cid, pl.ds(r, n)]))


_gate_cnt_call = pl.kernel(
    _gate_cnt_body,
    out_type=[
        jax.ShapeDtypeStruct((E, 16), jnp.float32),
        jax.ShapeDtypeStruct((E, 16), jnp.float32),
        jax.ShapeDtypeStruct((2, N, 128), jnp.float32),
    ],
    mesh=_mesh,
    scratch_types=[
        pltpu.VMEM((KG,), jnp.int32),
        pltpu.VMEM((KG,), jnp.int32),
        pltpu.VMEM((KG, HID), jnp.float32),
        pltpu.VMEM((KG, HID), jnp.float32),
        pltpu.VMEM((KG, 16), jnp.float32),
        pltpu.VMEM((KG, 16), jnp.float32),
        pltpu.VMEM((KG, 128), jnp.float32),
        pltpu.VMEM((8, 128), jnp.float32),
        pltpu.VMEM_SHARED((N, 128), jnp.float32),
        pltpu.SemaphoreType.DMA,
    ],
)

_agg_call = pl.kernel(
    _agg_body,
    out_type=jax.ShapeDtypeStruct((2, N, HID), jnp.float32),
    mesh=_mesh,
    scratch_types=[
        pltpu.VMEM((KE,), jnp.int32),
        pltpu.VMEM((KE,), jnp.int32),
        pltpu.VMEM((KE, HID), jnp.float32),
        pltpu.VMEM((KE, HID), jnp.float32),
        pltpu.VMEM((KE, 16), jnp.float32),
        pltpu.VMEM_SHARED((N, HID), jnp.float32),
        pltpu.SemaphoreType.DMA,
    ],
)


def _proj_in(ee, pe32, wt, wb, bi, wmt, a0, a1, b0, b1, g0, g1):
    nspec = pl.BlockSpec((BLK_N, EMB), lambda i: (i, 0))
    w3264 = pl.BlockSpec((32, 64), lambda i: (0, 0))
    b64 = pl.BlockSpec((1, 64), lambda i: (0, 0))
    return pl.pallas_call(
        _proj_in_kernel,
        grid=(N // BLK_N,),
        in_specs=[
            nspec,
            pl.BlockSpec((BLK_N, 32), lambda i: (i, 0)),
            pl.BlockSpec((EMB, HID), lambda i: (0, 0)),
            pl.BlockSpec((32, HID), lambda i: (0, 0)),
            pl.BlockSpec((1, HID), lambda i: (0, 0)),
            pl.BlockSpec((HID, HID), lambda i: (0, 0)),
            w3264, w3264, w3264, w3264, b64, b64,
        ],
        out_specs=[nspec, nspec, nspec, nspec],
        out_shape=[jax.ShapeDtypeStruct((N, HID), jnp.float32)] * 4,
    )(ee, pe32, wt, wb, bi, wmt, a0, a1, b0, b1, g0, g1)


def _rc(rel, wr, br, wm, bm):
    return pl.pallas_call(
        _rc_kernel,
        grid=(E // BLK_E,),
        in_specs=[
            pl.BlockSpec((BLK_E, EMB), lambda i: (i, 0)),
            pl.BlockSpec((EMB, REL), lambda i: (0, 0)),
            pl.BlockSpec((1, REL), lambda i: (0, 0)),
            pl.BlockSpec((REL, HID), lambda i: (0, 0)),
            pl.BlockSpec((1, HID), lambda i: (0, 0)),
        ],
        out_specs=pl.BlockSpec((BLK_E, HID), lambda i: (i, 0)),
        out_shape=jax.ShapeDtypeStruct((E, HID), jnp.float32),
    )(rel, wr, br, wm, bm)


def _upd(x, aggp, cnt16, wt, wb, b, wn, bn, final):
    nspec = pl.BlockSpec((BLK_N, HID), lambda i: (i, 0))
    wspec = pl.BlockSpec((HID, HID), lambda i: (0, 0))
    bspec = pl.BlockSpec((1, HID), lambda i: (0, 0))
    return pl.pallas_call(
        functools.partial(_upd_kernel, final=final),
        grid=(N // BLK_N,),
        in_specs=[
            nspec,
            pl.BlockSpec((2, BLK_N, HID), lambda i: (0, i, 0)),
            pl.BlockSpec((2, BLK_N, 128), lambda i: (0, i, 0)),
            wspec, wspec, bspec, wspec, bspec,
        ],
        out_specs=[nspec, nspec],
        out_shape=[jax.ShapeDtypeStruct((N, HID), jnp.float32)] * 2,
    )(x, aggp, cnt16, wt, wb, b, wn, bn)


def kernel(entity_embs, pe, edge_index, relation_embs_per_edge, W_in, b_in,
           W_rel, b_rel, W_msg, b_msg, W_g1, b_g1, W_g2, b_g2, W_upd, b_upd,
           W_out, b_out):
    src = edge_index[0]
    dst = edge_index[1]
    pe32 = jnp.pad(pe, ((0, 0), (0, 32 - PE)))

    w2e = jnp.zeros((8, 128), jnp.float32)
    w2e = w2e.at[:L, :64].set(W_g2[:, :, 0])
    w2e = w2e.at[:L, 64:80].set(jnp.broadcast_to(b_g2.reshape(L, 1), (L, 16)))

    x, xc, s_tab, d_tab = _proj_in(
        entity_embs, pe32, W_in[:EMB],
        jnp.pad(W_in[EMB:], ((0, 32 - PE), (0, 0))),
        b_in.reshape(1, HID), W_msg[0][:HID],
        jnp.pad(W_g1[0][:PE], ((0, 1), (0, 0))),
        jnp.pad(W_g1[1][:PE], ((0, 1), (0, 0))),
        jnp.pad(W_g1[0][PE:], ((0, 1), (0, 0))),
        jnp.pad(W_g1[1][PE:], ((0, 1), (0, 0))),
        b_g1[0].reshape(1, 64), b_g1[1].reshape(1, 64))

    rc0 = _rc(relation_embs_per_edge, W_rel, b_rel.reshape(1, REL),
              W_msg[0][HID:], b_msg[0].reshape(1, HID))
    rc1 = _rc(relation_embs_per_edge, W_rel, b_rel.reshape(1, REL),
              W_msg[1][HID:], b_msg[1].reshape(1, HID))

    gate0, gate1, cnt16 = _gate_cnt_call(src, dst, s_tab, d_tab, w2e)

    agg0p = _agg_call(src, dst, xc, rc0, gate0)
    x, xc = _upd(x, agg0p, cnt16, W_upd[0][:HID], W_upd[0][HID:],
                 b_upd[0].reshape(1, HID), W_msg[1][:HID],
                 jnp.zeros((1, HID), jnp.float32), final=False)

    agg1p = _agg_call(src, dst, xc, rc1, gate1)
    _, out = _upd(x, agg1p, cnt16, W_upd[1][:HID], W_upd[1][HID:],
                  b_upd[1].reshape(1, HID), W_out,
                  b_out.reshape(1, EMB), final=True)
    return out


# speedup vs baseline: 3.8690x; 1.1300x over previous
import functools

import jax
import jax.numpy as jnp
from jax import lax
from jax.experimental import pallas as pl
from jax.experimental.pallas import tpu as pltpu
from jax.experimental.pallas import tpu_sc as plsc

N = 10000
E = 320000
EMB = 128
HID = 128
REL = 64
PE = 31
L = 2

BLK_E = 512
BLK_N = 1000
TILE_ROWS = 624
_CHUNKS = ((0, 128), (128, 128), (256, 128), (384, 128), (512, 112))
_REM_OFF = 16 * TILE_ROWS
_REM = N - _REM_OFF

_mesh = plsc.VectorSubcoreMesh(core_axis_name="c", subcore_axis_name="s")


def _proj_in_kernel(ee_ref, pe_ref, wt_ref, wb_ref, bi_ref, wmt_ref,
                    a0_ref, a1_ref, b0_ref, b1_ref, g0_ref, g1_ref,
                    x_ref, xc_ref, s_ref, d_ref):
    pe = pe_ref[...]
    x = jnp.dot(ee_ref[...], wt_ref[...], preferred_element_type=jnp.float32)
    x = x + jnp.dot(pe, wb_ref[...], preferred_element_type=jnp.float32)
    x = jnp.maximum(x + bi_ref[...], 0.0)
    x_ref[...] = x
    xc_ref[...] = jnp.dot(x, wmt_ref[...], preferred_element_type=jnp.float32)
    s0 = jnp.dot(pe, a0_ref[...], preferred_element_type=jnp.float32)
    s1 = jnp.dot(pe, a1_ref[...], preferred_element_type=jnp.float32)
    s_ref[...] = jnp.concatenate([s0, s1], axis=1)
    d0 = jnp.dot(pe, b0_ref[...], preferred_element_type=jnp.float32)
    d1 = jnp.dot(pe, b1_ref[...], preferred_element_type=jnp.float32)
    d_ref[...] = jnp.concatenate([d0 + g0_ref[...], d1 + g1_ref[...]], axis=1)


def _rc_kernel(rel_ref, wr_ref, br_ref, wm_ref, bm_ref, rc_ref):
    ea = jnp.dot(rel_ref[...], wr_ref[...], preferred_element_type=jnp.float32)
    ea = ea + br_ref[...]
    rc = jnp.dot(ea, wm_ref[...], preferred_element_type=jnp.float32)
    rc_ref[...] = rc + bm_ref[...]


def _upd_kernel(x_ref, aggp_ref, cnt_ref, wt_ref, wb_ref, b_ref, wn_ref,
                bn_ref, x2_ref, xc2_ref, final):
    aggs = aggp_ref[0] + aggp_ref[1]
    cnt = jnp.maximum(cnt_ref[0, :, :1] + cnt_ref[1, :, :1], 1.0)
    agg = aggs / cnt
    x = jnp.dot(x_ref[...], wt_ref[...], preferred_element_type=jnp.float32)
    x = x + jnp.dot(agg, wb_ref[...], preferred_element_type=jnp.float32)
    x = jnp.maximum(x + b_ref[...], 0.0)
    x2_ref[...] = x
    y = jnp.dot(x, wn_ref[...], preferred_element_type=jnp.float32)
    if final:
        xc2_ref[...] = y + bn_ref[...]
    else:
        xc2_ref[...] = y


_GDN = lax.GatherDimensionNumbers(
    offset_dims=(), collapsed_slice_dims=(0,), start_index_map=(0,))


def _allsum(v):
    for s in (8, 4, 2, 1):
        idx = lax.iota(jnp.int32, 16) ^ s
        v = v + lax.gather(v, idx[:, None], _GDN, (1,),
                           mode=lax.GatherScatterMode.PROMISE_IN_BOUNDS)
    return v


def _tile_sweep(sid, copy_fn):
    row0 = sid * TILE_ROWS
    for off, nr in _CHUNKS:
        copy_fn(row0 + off, nr)

    @pl.when(sid == 0)
    def _():
        copy_fn(_REM_OFF, _REM)


KG = 64
NBLK_G = E // KG
MAX_WIN_G = (NBLK_G + 31) // 32
PAIRS_G = (MAX_WIN_G + 1) // 2


def _gate_from_rows(sg_v, dg_v, e, w2v, bg2v, l):
    acc = None
    for j in range(4):
        c = 64 * l + 16 * j
        h = jnp.maximum(sg_v[e, c:c + 16] + dg_v[e, c:c + 16], 0.0)
        t = h * w2v[4 * l + j]
        acc = t if acc is None else acc + t
    logit = _allsum(acc) + bg2v[l]
    return 1.0 / (1.0 + jnp.exp(-logit))


def _gate_cnt_body(src_hbm, dst_hbm, s_hbm, d_hbm, w2e_hbm,
                   gate0_hbm, gate1_hbm, cnt_hbm,
                   sidx, didx, sg_v, dg_v,
                   g0_v, g1_v, ones_v, w2_v, cnt_sh, sem):
    cid = lax.axis_index("c")
    sid = lax.axis_index("s")
    wid = sid * 2 + cid

    def fill(i, _):
        for j in range(8):
            ones_v[i, 16 * j:16 * (j + 1)] = jnp.zeros((16,), jnp.float32)
        return 0
    lax.fori_loop(0, KG, fill, 0)

    pltpu.sync_copy(w2e_hbm, w2_v)

    row0 = sid * TILE_ROWS
    for k in range(9):
        pltpu.sync_copy(ones_v, cnt_sh.at[pl.ds(row0 + 64 * k, 64)])
    pltpu.sync_copy(ones_v.at[pl.ds(0, 48)],
                    cnt_sh.at[pl.ds(row0 + 576, 48)])

    @pl.when(sid == 0)
    def _():
        pltpu.sync_copy(ones_v.at[pl.ds(0, _REM)],
                        cnt_sh.at[pl.ds(_REM_OFF, _REM)])

    def refill(i, _):
        for j in range(8):
            ones_v[i, 16 * j:16 * (j + 1)] = jnp.ones((16,), jnp.float32)
        return 0
    lax.fori_loop(0, KG, refill, 0)
    plsc.subcore_barrier()

    w2v = [w2_v[j // 4, 16 * (j % 4):16 * (j % 4) + 16] for j in range(8)]
    bg2v = [w2_v[l, 64:80] for l in range(L)]

    def body(i, _):
        b = wid + 32 * i

        @pl.when(b < NBLK_G)
        def _():
            base = b * KG
            pltpu.sync_copy(src_hbm.at[pl.ds(base, KG)], sidx)
            pltpu.sync_copy(dst_hbm.at[pl.ds(base, KG)], didx)
            cp2 = pltpu.async_copy(s_hbm.at[sidx], sg_v, sem)
            cp3 = pltpu.async_copy(d_hbm.at[didx], dg_v, sem)
            cp2.wait()
            cp3.wait()

            def ebody(ii, _):
                for u in range(2):
                    e = 2 * ii + u
                    g0_v[e, :] = _gate_from_rows(sg_v, dg_v, e, w2v, bg2v, 0)
                    g1_v[e, :] = _gate_from_rows(sg_v, dg_v, e, w2v, bg2v, 1)
                return 0
            lax.fori_loop(0, KG // 2, ebody, 0)

            pltpu.sync_copy(ones_v, cnt_sh.at[didx], add=True)
            pltpu.sync_copy(g0_v, gate0_hbm.at[pl.ds(base, KG)])
            pltpu.sync_copy(g1_v, gate1_hbm.at[pl.ds(base, KG)])
        return 0
    lax.fori_loop(0, MAX_WIN_G, body, 0)

    plsc.subcore_barrier()
    _tile_sweep(sid, lambda r, n: pltpu.sync_copy(
        cnt_sh.at[pl.ds(r, n)], cnt_hbm.at[cid, pl.ds(r, n)]))


def _agg_body(src_hbm, dst_hbm, xc_hbm, rc_hbm, gate_hbm,
              agg_hbm,
              sidx0, sidx1, didx0, didx1, xcg0, xcg1, rc0, rc1, gv0, gv1,
              agg_sh, sem0, sem1):
    cid = lax.axis_index("c")
    sid = lax.axis_index("s")
    wid = sid * 2 + cid
    sidx = (sidx0, sidx1)
    didx = (didx0, didx1)
    xcg = (xcg0, xcg1)
    rcv = (rc0, rc1)
    gv = (gv0, gv1)
    sem = (sem0, sem1)

    def fill(i, _):
        for j in range(8):
            rc0[i, 16 * j:16 * (j + 1)] = jnp.zeros((16,), jnp.float32)
        return 0
    lax.fori_loop(0, KG, fill, 0)

    row0 = sid * TILE_ROWS
    for k in range(9):
        pltpu.sync_copy(rc0, agg_sh.at[pl.ds(row0 + 64 * k, 64)])
    pltpu.sync_copy(rc0.at[pl.ds(0, 48)], agg_sh.at[pl.ds(row0 + 576, 48)])

    @pl.when(sid == 0)
    def _():
        pltpu.sync_copy(rc0.at[pl.ds(0, _REM)],
                        agg_sh.at[pl.ds(_REM_OFF, _REM)])
    plsc.subcore_barrier()

    def issue(s, b):
        base = b * KG
        pltpu.sync_copy(src_hbm.at[pl.ds(base, KG)], sidx[s])
        pltpu.sync_copy(dst_hbm.at[pl.ds(base, KG)], didx[s])
        pltpu.async_copy(xc_hbm.at[sidx[s]], xcg[s], sem[s])
        pltpu.async_copy(rc_hbm.at[pl.ds(base, KG)], rcv[s], sem[s])
        pltpu.async_copy(gate_hbm.at[pl.ds(base, KG)], gv[s], sem[s])

    def finish(s, b):
        base = b * KG
        pltpu.make_async_copy(xc_hbm.at[sidx[s]], xcg[s], sem[s]).wait()
        pltpu.make_async_copy(rc_hbm.at[pl.ds(base, KG)], rcv[s],
                              sem[s]).wait()
        pltpu.make_async_copy(gate_hbm.at[pl.ds(base, KG)], gv[s],
                              sem[s]).wait()

        def ebody(ii, _):
            for u in range(2):
                e = 2 * ii + u
                g = gv[s][e, :]
                for j in range(8):
                    sl = slice(16 * j, 16 * (j + 1))
                    rcv[s][e, sl] = g * jnp.maximum(
                        xcg[s][e, sl] + rcv[s][e, sl], 0.0)
            return 0
        lax.fori_loop(0, KG // 2, ebody, 0)

        pltpu.sync_copy(rcv[s], agg_sh.at[didx[s]], add=True)

    issue(0, wid)

    def body(k, _):
        b0 = wid + 32 * (2 * k)
        b1 = b0 + 32
        b2 = b0 + 64

        @pl.when(b1 < NBLK_G)
        def _():
            issue(1, b1)

        @pl.when(b0 < NBLK_G)
        def _():
            finish(0, b0)

        @pl.when(b2 < NBLK_G)
        def _():
            issue(0, b2)

        @pl.when(b1 < NBLK_G)
        def _():
            finish(1, b1)
        return 0
    lax.fori_loop(0, PAIRS_G, body, 0)

    plsc.subcore_barrier()
    _tile_sweep(sid, lambda r, n: pltpu.sync_copy(
        agg_sh.at[pl.ds(r, n)], agg_hbm.at[cid, pl.ds(r, n)]))


_gate_cnt_call = pl.kernel(
    _gate_cnt_body,
    out_type=[
        jax.ShapeDtypeStruct((E, 16), jnp.float32),
        jax.ShapeDtypeStruct((E, 16), jnp.float32),
        jax.ShapeDtypeStruct((2, N, 128), jnp.float32),
    ],
    mesh=_mesh,
    scratch_types=[
        pltpu.VMEM((KG,), jnp.int32),
        pltpu.VMEM((KG,), jnp.int32),
        pltpu.VMEM((KG, HID), jnp.float32),
        pltpu.VMEM((KG, HID), jnp.float32),
        pltpu.VMEM((KG, 16), jnp.float32),
        pltpu.VMEM((KG, 16), jnp.float32),
        pltpu.VMEM((KG, 128), jnp.float32),
        pltpu.VMEM((8, 128), jnp.float32),
        pltpu.VMEM_SHARED((N, 128), jnp.float32),
        pltpu.SemaphoreType.DMA,
    ],
)

_agg_call = pl.kernel(
    _agg_body,
    out_type=jax.ShapeDtypeStruct((2, N, HID), jnp.float32),
    mesh=_mesh,
    scratch_types=[
        pltpu.VMEM((KG,), jnp.int32),
        pltpu.VMEM((KG,), jnp.int32),
        pltpu.VMEM((KG,), jnp.int32),
        pltpu.VMEM((KG,), jnp.int32),
        pltpu.VMEM((KG, HID), jnp.float32),
        pltpu.VMEM((KG, HID), jnp.float32),
        pltpu.VMEM((KG, HID), jnp.float32),
        pltpu.VMEM((KG, HID), jnp.float32),
        pltpu.VMEM((KG, 16), jnp.float32),
        pltpu.VMEM((KG, 16), jnp.float32),
        pltpu.VMEM_SHARED((N, HID), jnp.float32),
        pltpu.SemaphoreType.DMA,
        pltpu.SemaphoreType.DMA,
    ],
)


def _proj_in(ee, pe32, wt, wb, bi, wmt, a0, a1, b0, b1, g0, g1):
    nspec = pl.BlockSpec((BLK_N, EMB), lambda i: (i, 0))
    w3264 = pl.BlockSpec((32, 64), lambda i: (0, 0))
    b64 = pl.BlockSpec((1, 64), lambda i: (0, 0))
    return pl.pallas_call(
        _proj_in_kernel,
        grid=(N // BLK_N,),
        in_specs=[
            nspec,
            pl.BlockSpec((BLK_N, 32), lambda i: (i, 0)),
            pl.BlockSpec((EMB, HID), lambda i: (0, 0)),
            pl.BlockSpec((32, HID), lambda i: (0, 0)),
            pl.BlockSpec((1, HID), lambda i: (0, 0)),
            pl.BlockSpec((HID, HID), lambda i: (0, 0)),
            w3264, w3264, w3264, w3264, b64, b64,
        ],
        out_specs=[nspec, nspec, nspec, nspec],
        out_shape=[jax.ShapeDtypeStruct((N, HID), jnp.float32)] * 4,
    )(ee, pe32, wt, wb, bi, wmt, a0, a1, b0, b1, g0, g1)


def _rc(rel, wr, br, wm, bm):
    return pl.pallas_call(
        _rc_kernel,
        grid=(E // BLK_E,),
        in_specs=[
            pl.BlockSpec((BLK_E, EMB), lambda i: (i, 0)),
            pl.BlockSpec((EMB, REL), lambda i: (0, 0)),
            pl.BlockSpec((1, REL), lambda i: (0, 0)),
            pl.BlockSpec((REL, HID), lambda i: (0, 0)),
            pl.BlockSpec((1, HID), lambda i: (0, 0)),
        ],
        out_specs=pl.BlockSpec((BLK_E, HID), lambda i: (i, 0)),
        out_shape=jax.ShapeDtypeStruct((E, HID), jnp.float32),
    )(rel, wr, br, wm, bm)


def _upd(x, aggp, cnt16, wt, wb, b, wn, bn, final):
    nspec = pl.BlockSpec((BLK_N, HID), lambda i: (i, 0))
    wspec = pl.BlockSpec((HID, HID), lambda i: (0, 0))
    bspec = pl.BlockSpec((1, HID), lambda i: (0, 0))
    return pl.pallas_call(
        functools.partial(_upd_kernel, final=final),
        grid=(N // BLK_N,),
        in_specs=[
            nspec,
            pl.BlockSpec((2, BLK_N, HID), lambda i: (0, i, 0)),
            pl.BlockSpec((2, BLK_N, 128), lambda i: (0, i, 0)),
            wspec, wspec, bspec, wspec, bspec,
        ],
        out_specs=[nspec, nspec],
        out_shape=[jax.ShapeDtypeStruct((N, HID), jnp.float32)] * 2,
    )(x, aggp, cnt16, wt, wb, b, wn, bn)


def kernel(entity_embs, pe, edge_index, relation_embs_per_edge, W_in, b_in,
           W_rel, b_rel, W_msg, b_msg, W_g1, b_g1, W_g2, b_g2, W_upd, b_upd,
           W_out, b_out):
    src = edge_index[0]
    dst = edge_index[1]
    pe32 = jnp.pad(pe, ((0, 0), (0, 32 - PE)))

    w2e = jnp.zeros((8, 128), jnp.float32)
    w2e = w2e.at[:L, :64].set(W_g2[:, :, 0])
    w2e = w2e.at[:L, 64:80].set(jnp.broadcast_to(b_g2.reshape(L, 1), (L, 16)))

    x, xc, s_tab, d_tab = _proj_in(
        entity_embs, pe32, W_in[:EMB],
        jnp.pad(W_in[EMB:], ((0, 32 - PE), (0, 0))),
        b_in.reshape(1, HID), W_msg[0][:HID],
        jnp.pad(W_g1[0][:PE], ((0, 1), (0, 0))),
        jnp.pad(W_g1[1][:PE], ((0, 1), (0, 0))),
        jnp.pad(W_g1[0][PE:], ((0, 1), (0, 0))),
        jnp.pad(W_g1[1][PE:], ((0, 1), (0, 0))),
        b_g1[0].reshape(1, 64), b_g1[1].reshape(1, 64))

    rc0 = _rc(relation_embs_per_edge, W_rel, b_rel.reshape(1, REL),
              W_msg[0][HID:], b_msg[0].reshape(1, HID))
    rc1 = _rc(relation_embs_per_edge, W_rel, b_rel.reshape(1, REL),
              W_msg[1][HID:], b_msg[1].reshape(1, HID))

    gate0, gate1, cnt16 = _gate_cnt_call(src, dst, s_tab, d_tab, w2e)

    agg0p = _agg_call(src, dst, xc, rc0, gate0)
    x, xc = _upd(x, agg0p, cnt16, W_upd[0][:HID], W_upd[0][HID:],
                 b_upd[0].reshape(1, HID), W_msg[1][:HID],
                 jnp.zeros((1, HID), jnp.float32), final=False)

    agg1p = _agg_call(src, dst, xc, rc1, gate1)
    _, out = _upd(x, agg1p, cnt16, W_upd[1][:HID], W_upd[1][HID:],
                  b_upd[1].reshape(1, HID), W_out,
                  b_out.reshape(1, EMB), final=True)
    return out


# speedup vs baseline: 3.8716x; 1.0007x over previous
import functools

import jax
import jax.numpy as jnp
from jax import lax
from jax.experimental import pallas as pl
from jax.experimental.pallas import tpu as pltpu
from jax.experimental.pallas import tpu_sc as plsc

N = 10000
E = 320000
EMB = 128
HID = 128
REL = 64
PE = 31
L = 2

BLK_E = 512
BLK_N = 1000
TILE_ROWS = 624
_CHUNKS = ((0, 128), (128, 128), (256, 128), (384, 128), (512, 112))
_REM_OFF = 16 * TILE_ROWS
_REM = N - _REM_OFF

_mesh = plsc.VectorSubcoreMesh(core_axis_name="c", subcore_axis_name="s")


def _proj_in_kernel(ee_ref, pe_ref, wt_ref, wb_ref, bi_ref, wmt_ref,
                    a0_ref, a1_ref, b0_ref, b1_ref, g0_ref, g1_ref,
                    x_ref, xc_ref, s_ref, d_ref):
    pe = pe_ref[...]
    x = jnp.dot(ee_ref[...], wt_ref[...], preferred_element_type=jnp.float32)
    x = x + jnp.dot(pe, wb_ref[...], preferred_element_type=jnp.float32)
    x = jnp.maximum(x + bi_ref[...], 0.0)
    x_ref[...] = x
    xc_ref[...] = jnp.dot(x, wmt_ref[...], preferred_element_type=jnp.float32)
    s0 = jnp.dot(pe, a0_ref[...], preferred_element_type=jnp.float32)
    s1 = jnp.dot(pe, a1_ref[...], preferred_element_type=jnp.float32)
    s_ref[...] = jnp.concatenate([s0, s1], axis=1)
    d0 = jnp.dot(pe, b0_ref[...], preferred_element_type=jnp.float32)
    d1 = jnp.dot(pe, b1_ref[...], preferred_element_type=jnp.float32)
    d_ref[...] = jnp.concatenate([d0 + g0_ref[...], d1 + g1_ref[...]], axis=1)


def _rc_kernel(rel_ref, wr_ref, br_ref, wm_ref, bm_ref, rc_ref):
    ea = jnp.dot(rel_ref[...], wr_ref[...], preferred_element_type=jnp.float32)
    ea = ea + br_ref[...]
    rc = jnp.dot(ea, wm_ref[...], preferred_element_type=jnp.float32)
    rc_ref[...] = rc + bm_ref[...]


def _upd_kernel(x_ref, aggp_ref, cnt_ref, wt_ref, wb_ref, b_ref, wn_ref,
                bn_ref, x2_ref, xc2_ref, final):
    aggs = aggp_ref[0] + aggp_ref[1]
    cnt = jnp.maximum(cnt_ref[0, :, :1] + cnt_ref[1, :, :1], 1.0)
    agg = aggs / cnt
    x = jnp.dot(x_ref[...], wt_ref[...], preferred_element_type=jnp.float32)
    x = x + jnp.dot(agg, wb_ref[...], preferred_element_type=jnp.float32)
    x = jnp.maximum(x + b_ref[...], 0.0)
    x2_ref[...] = x
    y = jnp.dot(x, wn_ref[...], preferred_element_type=jnp.float32)
    if final:
        xc2_ref[...] = y + bn_ref[...]
    else:
        xc2_ref[...] = y


_GDN = lax.GatherDimensionNumbers(
    offset_dims=(), collapsed_slice_dims=(0,), start_index_map=(0,))


def _allsum(v):
    for s in (8, 4, 2, 1):
        idx = lax.iota(jnp.int32, 16) ^ s
        v = v + lax.gather(v, idx[:, None], _GDN, (1,),
                           mode=lax.GatherScatterMode.PROMISE_IN_BOUNDS)
    return v


def _tile_sweep(sid, copy_fn):
    row0 = sid * TILE_ROWS
    for off, nr in _CHUNKS:
        copy_fn(row0 + off, nr)

    @pl.when(sid == 0)
    def _():
        copy_fn(_REM_OFF, _REM)


KG = 64
NBLK_G = E // KG
MAX_WIN_G = (NBLK_G + 31) // 32
PAIRS_G = (MAX_WIN_G + 1) // 2


def _gate_from_rows(sg_v, dg_v, e, w2v, bg2v, l):
    acc = None
    for j in range(4):
        c = 64 * l + 16 * j
        h = jnp.maximum(sg_v[e, c:c + 16] + dg_v[e, c:c + 16], 0.0)
        t = h * w2v[4 * l + j]
        acc = t if acc is None else acc + t
    logit = _allsum(acc) + bg2v[l]
    return 1.0 / (1.0 + jnp.exp(-logit))


def _gate_cnt_body(src_hbm, dst_hbm, s_hbm, d_hbm, w2e_hbm,
                   gate0_hbm, gate1_hbm, cnt_hbm,
                   sidx, didx, sg_v, dg_v,
                   g0_v, g1_v, ones_v, w2_v, cnt_sh, sem):
    cid = lax.axis_index("c")
    sid = lax.axis_index("s")
    wid = sid * 2 + cid

    def fill(i, _):
        for j in range(8):
            ones_v[i, 16 * j:16 * (j + 1)] = jnp.zeros((16,), jnp.float32)
        return 0
    lax.fori_loop(0, KG, fill, 0)

    pltpu.sync_copy(w2e_hbm, w2_v)

    row0 = sid * TILE_ROWS
    for k in range(9):
        pltpu.sync_copy(ones_v, cnt_sh.at[pl.ds(row0 + 64 * k, 64)])
    pltpu.sync_copy(ones_v.at[pl.ds(0, 48)],
                    cnt_sh.at[pl.ds(row0 + 576, 48)])

    @pl.when(sid == 0)
    def _():
        pltpu.sync_copy(ones_v.at[pl.ds(0, _REM)],
                        cnt_sh.at[pl.ds(_REM_OFF, _REM)])

    def refill(i, _):
        for j in range(8):
            ones_v[i, 16 * j:16 * (j + 1)] = jnp.ones((16,), jnp.float32)
        return 0
    lax.fori_loop(0, KG, refill, 0)
    plsc.subcore_barrier()

    w2v = [w2_v[j // 4, 16 * (j % 4):16 * (j % 4) + 16] for j in range(8)]
    bg2v = [w2_v[l, 64:80] for l in range(L)]

    def body(i, _):
        b = wid + 32 * i

        @pl.when(b < NBLK_G)
        def _():
            base = b * KG
            pltpu.sync_copy(src_hbm.at[pl.ds(base, KG)], sidx)
            pltpu.sync_copy(dst_hbm.at[pl.ds(base, KG)], didx)
            cp2 = pltpu.async_copy(s_hbm.at[sidx], sg_v, sem)
            cp3 = pltpu.async_copy(d_hbm.at[didx], dg_v, sem)
            cp2.wait()
            cp3.wait()

            def ebody(ii, _):
                for u in range(4):
                    e = 4 * ii + u
                    g0_v[e, :] = _gate_from_rows(sg_v, dg_v, e, w2v, bg2v, 0)
                    g1_v[e, :] = _gate_from_rows(sg_v, dg_v, e, w2v, bg2v, 1)
                return 0
            lax.fori_loop(0, KG // 4, ebody, 0)

            pltpu.sync_copy(ones_v, cnt_sh.at[didx], add=True)
            pltpu.sync_copy(g0_v, gate0_hbm.at[pl.ds(base, KG)])
            pltpu.sync_copy(g1_v, gate1_hbm.at[pl.ds(base, KG)])
        return 0
    lax.fori_loop(0, MAX_WIN_G, body, 0)

    plsc.subcore_barrier()
    _tile_sweep(sid, lambda r, n: pltpu.sync_copy(
        cnt_sh.at[pl.ds(r, n)], cnt_hbm.at[cid, pl.ds(r, n)]))


def _agg_body(src_hbm, dst_hbm, xc_hbm, rc_hbm, gate_hbm,
              agg_hbm,
              sidx0, sidx1, didx0, didx1, xcg0, xcg1, rc0, rc1, gv0, gv1,
              agg_sh, sem0, sem1):
    cid = lax.axis_index("c")
    sid = lax.axis_index("s")
    wid = sid * 2 + cid
    sidx = (sidx0, sidx1)
    didx = (didx0, didx1)
    xcg = (xcg0, xcg1)
    rcv = (rc0, rc1)
    gv = (gv0, gv1)
    sem = (sem0, sem1)

    def fill(i, _):
        for j in range(8):
            rc0[i, 16 * j:16 * (j + 1)] = jnp.zeros((16,), jnp.float32)
        return 0
    lax.fori_loop(0, KG, fill, 0)

    row0 = sid * TILE_ROWS
    for k in range(9):
        pltpu.sync_copy(rc0, agg_sh.at[pl.ds(row0 + 64 * k, 64)])
    pltpu.sync_copy(rc0.at[pl.ds(0, 48)], agg_sh.at[pl.ds(row0 + 576, 48)])

    @pl.when(sid == 0)
    def _():
        pltpu.sync_copy(rc0.at[pl.ds(0, _REM)],
                        agg_sh.at[pl.ds(_REM_OFF, _REM)])
    plsc.subcore_barrier()

    def issue(s, b):
        base = b * KG
        pltpu.sync_copy(src_hbm.at[pl.ds(base, KG)], sidx[s])
        pltpu.sync_copy(dst_hbm.at[pl.ds(base, KG)], didx[s])
        pltpu.async_copy(xc_hbm.at[sidx[s]], xcg[s], sem[s])
        pltpu.async_copy(rc_hbm.at[pl.ds(base, KG)], rcv[s], sem[s])
        pltpu.async_copy(gate_hbm.at[pl.ds(base, KG)], gv[s], sem[s])

    def finish(s, b):
        base = b * KG
        pltpu.make_async_copy(xc_hbm.at[sidx[s]], xcg[s], sem[s]).wait()
        pltpu.make_async_copy(rc_hbm.at[pl.ds(base, KG)], rcv[s],
                              sem[s]).wait()
        pltpu.make_async_copy(gate_hbm.at[pl.ds(base, KG)], gv[s],
                              sem[s]).wait()

        def ebody(ii, _):
            for u in range(4):
                e = 4 * ii + u
                g = gv[s][e, :]
                for j in range(8):
                    sl = slice(16 * j, 16 * (j + 1))
                    rcv[s][e, sl] = g * jnp.maximum(
                        xcg[s][e, sl] + rcv[s][e, sl], 0.0)
            return 0
        lax.fori_loop(0, KG // 4, ebody, 0)

        pltpu.sync_copy(rcv[s], agg_sh.at[didx[s]], add=True)

    issue(0, wid)

    def body(k, _):
        b0 = wid + 32 * (2 * k)
        b1 = b0 + 32
        b2 = b0 + 64

        @pl.when(b1 < NBLK_G)
        def _():
            issue(1, b1)

        @pl.when(b0 < NBLK_G)
        def _():
            finish(0, b0)

        @pl.when(b2 < NBLK_G)
        def _():
            issue(0, b2)

        @pl.when(b1 < NBLK_G)
        def _():
            finish(1, b1)
        return 0
    lax.fori_loop(0, PAIRS_G, body, 0)

    plsc.subcore_barrier()
    _tile_sweep(sid, lambda r, n: pltpu.sync_copy(
        agg_sh.at[pl.ds(r, n)], agg_hbm.at[cid, pl.ds(r, n)]))


_gate_cnt_call = pl.kernel(
    _gate_cnt_body,
    out_type=[
        jax.ShapeDtypeStruct((E, 16), jnp.float32),
        jax.ShapeDtypeStruct((E, 16), jnp.float32),
        jax.ShapeDtypeStruct((2, N, 128), jnp.float32),
    ],
    mesh=_mesh,
    scratch_types=[
        pltpu.VMEM((KG,), jnp.int32),
        pltpu.VMEM((KG,), jnp.int32),
        pltpu.VMEM((KG, HID), jnp.float32),
        pltpu.VMEM((KG, HID), jnp.float32),
        pltpu.VMEM((KG, 16), jnp.float32),
        pltpu.VMEM((KG, 16), jnp.float32),
        pltpu.VMEM((KG, 128), jnp.float32),
        pltpu.VMEM((8, 128), jnp.float32),
        pltpu.VMEM_SHARED((N, 128), jnp.float32),
        pltpu.SemaphoreType.DMA,
    ],
)

_agg_call = pl.kernel(
    _agg_body,
    out_type=jax.ShapeDtypeStruct((2, N, HID), jnp.float32),
    mesh=_mesh,
    scratch_types=[
        pltpu.VMEM((KG,), jnp.int32),
        pltpu.VMEM((KG,), jnp.int32),
        pltpu.VMEM((KG,), jnp.int32),
        pltpu.VMEM((KG,), jnp.int32),
        pltpu.VMEM((KG, HID), jnp.float32),
        pltpu.VMEM((KG, HID), jnp.float32),
        pltpu.VMEM((KG, HID), jnp.float32),
        pltpu.VMEM((KG, HID), jnp.float32),
        pltpu.VMEM((KG, 16), jnp.float32),
        pltpu.VMEM((KG, 16), jnp.float32),
        pltpu.VMEM_SHARED((N, HID), jnp.float32),
        pltpu.SemaphoreType.DMA,
        pltpu.SemaphoreType.DMA,
    ],
)


def _proj_in(ee, pe32, wt, wb, bi, wmt, a0, a1, b0, b1, g0, g1):
    nspec = pl.BlockSpec((BLK_N, EMB), lambda i: (i, 0))
    w3264 = pl.BlockSpec((32, 64), lambda i: (0, 0))
    b64 = pl.BlockSpec((1, 64), lambda i: (0, 0))
    return pl.pallas_call(
        _proj_in_kernel,
        grid=(N // BLK_N,),
        in_specs=[
            nspec,
            pl.BlockSpec((BLK_N, 32), lambda i: (i, 0)),
            pl.BlockSpec((EMB, HID), lambda i: (0, 0)),
            pl.BlockSpec((32, HID), lambda i: (0, 0)),
            pl.BlockSpec((1, HID), lambda i: (0, 0)),
            pl.BlockSpec((HID, HID), lambda i: (0, 0)),
            w3264, w3264, w3264, w3264, b64, b64,
        ],
        out_specs=[nspec, nspec, nspec, nspec],
        out_shape=[jax.ShapeDtypeStruct((N, HID), jnp.float32)] * 4,
    )(ee, pe32, wt, wb, bi, wmt, a0, a1, b0, b1, g0, g1)


def _rc(rel, wr, br, wm, bm):
    return pl.pallas_call(
        _rc_kernel,
        grid=(E // BLK_E,),
        in_specs=[
            pl.BlockSpec((BLK_E, EMB), lambda i: (i, 0)),
            pl.BlockSpec((EMB, REL), lambda i: (0, 0)),
            pl.BlockSpec((1, REL), lambda i: (0, 0)),
            pl.BlockSpec((REL, HID), lambda i: (0, 0)),
            pl.BlockSpec((1, HID), lambda i: (0, 0)),
        ],
        out_specs=pl.BlockSpec((BLK_E, HID), lambda i: (i, 0)),
        out_shape=jax.ShapeDtypeStruct((E, HID), jnp.float32),
    )(rel, wr, br, wm, bm)


def _upd(x, aggp, cnt16, wt, wb, b, wn, bn, final):
    nspec = pl.BlockSpec((BLK_N, HID), lambda i: (i, 0))
    wspec = pl.BlockSpec((HID, HID), lambda i: (0, 0))
    bspec = pl.BlockSpec((1, HID), lambda i: (0, 0))
    return pl.pallas_call(
        functools.partial(_upd_kernel, final=final),
        grid=(N // BLK_N,),
        in_specs=[
            nspec,
            pl.BlockSpec((2, BLK_N, HID), lambda i: (0, i, 0)),
            pl.BlockSpec((2, BLK_N, 128), lambda i: (0, i, 0)),
            wspec, wspec, bspec, wspec, bspec,
        ],
        out_specs=[nspec, nspec],
        out_shape=[jax.ShapeDtypeStruct((N, HID), jnp.float32)] * 2,
    )(x, aggp, cnt16, wt, wb, b, wn, bn)


def kernel(entity_embs, pe, edge_index, relation_embs_per_edge, W_in, b_in,
           W_rel, b_rel, W_msg, b_msg, W_g1, b_g1, W_g2, b_g2, W_upd, b_upd,
           W_out, b_out):
    src = edge_index[0]
    dst = edge_index[1]
    pe32 = jnp.pad(pe, ((0, 0), (0, 32 - PE)))

    w2e = jnp.zeros((8, 128), jnp.float32)
    w2e = w2e.at[:L, :64].set(W_g2[:, :, 0])
    w2e = w2e.at[:L, 64:80].set(jnp.broadcast_to(b_g2.reshape(L, 1), (L, 16)))

    x, xc, s_tab, d_tab = _proj_in(
        entity_embs, pe32, W_in[:EMB],
        jnp.pad(W_in[EMB:], ((0, 32 - PE), (0, 0))),
        b_in.reshape(1, HID), W_msg[0][:HID],
        jnp.pad(W_g1[0][:PE], ((0, 1), (0, 0))),
        jnp.pad(W_g1[1][:PE], ((0, 1), (0, 0))),
        jnp.pad(W_g1[0][PE:], ((0, 1), (0, 0))),
        jnp.pad(W_g1[1][PE:], ((0, 1), (0, 0))),
        b_g1[0].reshape(1, 64), b_g1[1].reshape(1, 64))

    gate0, gate1, cnt16 = _gate_cnt_call(src, dst, s_tab, d_tab, w2e)

    rc0 = _rc(relation_embs_per_edge, W_rel, b_rel.reshape(1, REL),
              W_msg[0][HID:], b_msg[0].reshape(1, HID))
    rc1 = _rc(relation_embs_per_edge, W_rel, b_rel.reshape(1, REL),
              W_msg[1][HID:], b_msg[1].reshape(1, HID))

    agg0p = _agg_call(src, dst, xc, rc0, gate0)
    x, xc = _upd(x, agg0p, cnt16, W_upd[0][:HID], W_upd[0][HID:],
                 b_upd[0].reshape(1, HID), W_msg[1][:HID],
                 jnp.zeros((1, HID), jnp.float32), final=False)

    agg1p = _agg_call(src, dst, xc, rc1, gate1)
    _, out = _upd(x, agg1p, cnt16, W_upd[1][:HID], W_upd[1][HID:],
                  b_upd[1].reshape(1, HID), W_out,
                  b_out.reshape(1, EMB), final=True)
    return out


# speedup vs baseline: 4.1725x; 1.0777x over previous
import functools

import jax
import jax.numpy as jnp
from jax import lax
from jax.experimental import pallas as pl
from jax.experimental.pallas import tpu as pltpu
from jax.experimental.pallas import tpu_sc as plsc

N = 10000
E = 320000
EMB = 128
HID = 128
REL = 64
PE = 31
L = 2

BLK_E = 512
BLK_N = 1000
TILE_ROWS = 624
_CHUNKS = ((0, 128), (128, 128), (256, 128), (384, 128), (512, 112))
_REM_OFF = 16 * TILE_ROWS
_REM = N - _REM_OFF

_mesh = plsc.VectorSubcoreMesh(core_axis_name="c", subcore_axis_name="s")


def _proj_in_kernel(ee_ref, pe_ref, wt_ref, wb_ref, bi_ref, wmt_ref,
                    a0_ref, a1_ref, b0_ref, b1_ref, g0_ref, g1_ref,
                    x_ref, xc_ref, s_ref, d_ref):
    pe = pe_ref[...]
    x = jnp.dot(ee_ref[...], wt_ref[...], preferred_element_type=jnp.float32)
    x = x + jnp.dot(pe, wb_ref[...], preferred_element_type=jnp.float32)
    x = jnp.maximum(x + bi_ref[...], 0.0)
    x_ref[...] = x
    xc_ref[...] = jnp.dot(x, wmt_ref[...], preferred_element_type=jnp.float32)
    s0 = jnp.dot(pe, a0_ref[...], preferred_element_type=jnp.float32)
    s1 = jnp.dot(pe, a1_ref[...], preferred_element_type=jnp.float32)
    s_ref[...] = jnp.concatenate([s0, s1], axis=1)
    d0 = jnp.dot(pe, b0_ref[...], preferred_element_type=jnp.float32)
    d1 = jnp.dot(pe, b1_ref[...], preferred_element_type=jnp.float32)
    d_ref[...] = jnp.concatenate([d0 + g0_ref[...], d1 + g1_ref[...]], axis=1)


def _rc_kernel(rel_ref, wr_ref, br_ref, wm_ref, bm_ref, rc_ref):
    ea = jnp.dot(rel_ref[...], wr_ref[...], preferred_element_type=jnp.float32)
    ea = ea + br_ref[...]
    rc = jnp.dot(ea, wm_ref[...], preferred_element_type=jnp.float32)
    rc_ref[...] = rc + bm_ref[...]


def _upd_kernel(x_ref, aggp_ref, cnt_ref, wt_ref, wb_ref, b_ref, wn_ref,
                bn_ref, x2_ref, xc2_ref, final):
    aggs = aggp_ref[0] + aggp_ref[1]
    cnt = jnp.maximum(cnt_ref[0, :, :1] + cnt_ref[1, :, :1], 1.0)
    agg = aggs / cnt
    x = jnp.dot(x_ref[...], wt_ref[...], preferred_element_type=jnp.float32)
    x = x + jnp.dot(agg, wb_ref[...], preferred_element_type=jnp.float32)
    x = jnp.maximum(x + b_ref[...], 0.0)
    x2_ref[...] = x
    y = jnp.dot(x, wn_ref[...], preferred_element_type=jnp.float32)
    if final:
        xc2_ref[...] = y + bn_ref[...]
    else:
        xc2_ref[...] = y


_GDN = lax.GatherDimensionNumbers(
    offset_dims=(), collapsed_slice_dims=(0,), start_index_map=(0,))


def _allsum(v):
    for s in (8, 4, 2, 1):
        idx = lax.iota(jnp.int32, 16) ^ s
        v = v + lax.gather(v, idx[:, None], _GDN, (1,),
                           mode=lax.GatherScatterMode.PROMISE_IN_BOUNDS)
    return v


def _tile_sweep(sid, copy_fn):
    row0 = sid * TILE_ROWS
    for off, nr in _CHUNKS:
        copy_fn(row0 + off, nr)

    @pl.when(sid == 0)
    def _():
        copy_fn(_REM_OFF, _REM)


KG = 64
NBLK_G = E // KG
MAX_WIN_G = (NBLK_G + 31) // 32
PAIRS_G = (MAX_WIN_G + 1) // 2
KW = 80
NBLK_W = E // KW
MAX_WIN_W = (NBLK_W + 31) // 32
PAIRS_W = (MAX_WIN_W + 1) // 2


def _gate_from_rows(sg_v, dg_v, e, w2v, bg2v, l):
    acc = None
    for j in range(4):
        c = 64 * l + 16 * j
        h = jnp.maximum(sg_v[e, c:c + 16] + dg_v[e, c:c + 16], 0.0)
        t = h * w2v[4 * l + j]
        acc = t if acc is None else acc + t
    logit = _allsum(acc) + bg2v[l]
    return 1.0 / (1.0 + jnp.exp(-logit))


def _gate_body(src_hbm, dst_hbm, s_hbm, d_hbm, w2e_hbm,
                   gate0_hbm, gate1_hbm,
                   sidx0, sidx1, didx0, didx1, sg0, sg1, dg0, dg1,
                   g00, g01, g10, g11, w2_v, sem0, sem1):
    cid = lax.axis_index("c")
    sid = lax.axis_index("s")
    wid = sid * 2 + cid
    sidx = (sidx0, sidx1)
    didx = (didx0, didx1)
    sg = (sg0, sg1)
    dg = (dg0, dg1)
    g0_v = (g00, g01)
    g1_v = (g10, g11)
    sem = (sem0, sem1)

    pltpu.sync_copy(w2e_hbm, w2_v)

    w2v = [w2_v[j // 4, 16 * (j % 4):16 * (j % 4) + 16] for j in range(8)]
    bg2v = [w2_v[l, 64:80] for l in range(L)]

    def issue(s, b):
        base = b * KW
        pltpu.sync_copy(src_hbm.at[pl.ds(base, KW)], sidx[s])
        pltpu.sync_copy(dst_hbm.at[pl.ds(base, KW)], didx[s])
        pltpu.async_copy(s_hbm.at[sidx[s]], sg[s], sem[s])
        pltpu.async_copy(d_hbm.at[didx[s]], dg[s], sem[s])

    def finish(s, b):
        base = b * KW
        pltpu.make_async_copy(s_hbm.at[sidx[s]], sg[s], sem[s]).wait()
        pltpu.make_async_copy(d_hbm.at[didx[s]], dg[s], sem[s]).wait()

        def ebody(ii, _):
            for u in range(2):
                e = 2 * ii + u
                g0_v[s][e, :] = _gate_from_rows(sg[s], dg[s], e, w2v, bg2v, 0)
                g1_v[s][e, :] = _gate_from_rows(sg[s], dg[s], e, w2v, bg2v, 1)
            return 0
        lax.fori_loop(0, KW // 2, ebody, 0)

        pltpu.sync_copy(g0_v[s], gate0_hbm.at[pl.ds(base, KW)])
        pltpu.sync_copy(g1_v[s], gate1_hbm.at[pl.ds(base, KW)])

    issue(0, wid)

    def body(k, _):
        b0 = wid + 32 * (2 * k)
        b1 = b0 + 32
        b2 = b0 + 64

        @pl.when(b1 < NBLK_W)
        def _():
            issue(1, b1)

        @pl.when(b0 < NBLK_W)
        def _():
            finish(0, b0)

        @pl.when(b2 < NBLK_W)
        def _():
            issue(0, b2)

        @pl.when(b1 < NBLK_W)
        def _():
            finish(1, b1)
        return 0
    lax.fori_loop(0, PAIRS_W, body, 0)


def _cnt_body(dst_hbm, cnt_hbm,
              didx0, didx1, ones_v, cnt_sh, sem0, sem1):
    cid = lax.axis_index("c")
    sid = lax.axis_index("s")
    wid = sid * 2 + cid
    didx = (didx0, didx1)
    sem = (sem0, sem1)

    def fill(i, _):
        for j in range(8):
            ones_v[i, 16 * j:16 * (j + 1)] = jnp.zeros((16,), jnp.float32)
        return 0
    lax.fori_loop(0, KW, fill, 0)

    row0 = sid * TILE_ROWS
    for k in range(7):
        pltpu.sync_copy(ones_v, cnt_sh.at[pl.ds(row0 + 80 * k, 80)])
    pltpu.sync_copy(ones_v.at[pl.ds(0, 64)],
                    cnt_sh.at[pl.ds(row0 + 560, 64)])

    @pl.when(sid == 0)
    def _():
        pltpu.sync_copy(ones_v.at[pl.ds(0, _REM)],
                        cnt_sh.at[pl.ds(_REM_OFF, _REM)])

    def refill(i, _):
        for j in range(8):
            ones_v[i, 16 * j:16 * (j + 1)] = jnp.ones((16,), jnp.float32)
        return 0
    lax.fori_loop(0, KW, refill, 0)
    plsc.subcore_barrier()

    def issue(s, b):
        pltpu.async_copy(dst_hbm.at[pl.ds(b * KW, KW)], didx[s], sem[s])

    def finish(s, b):
        pltpu.make_async_copy(dst_hbm.at[pl.ds(b * KW, KW)], didx[s],
                              sem[s]).wait()
        pltpu.sync_copy(ones_v, cnt_sh.at[didx[s]], add=True)

    issue(0, wid)

    def body(k, _):
        b0 = wid + 32 * (2 * k)
        b1 = b0 + 32
        b2 = b0 + 64

        @pl.when(b1 < NBLK_W)
        def _():
            issue(1, b1)

        @pl.when(b0 < NBLK_W)
        def _():
            finish(0, b0)

        @pl.when(b2 < NBLK_W)
        def _():
            issue(0, b2)

        @pl.when(b1 < NBLK_W)
        def _():
            finish(1, b1)
        return 0
    lax.fori_loop(0, PAIRS_W, body, 0)

    plsc.subcore_barrier()
    _tile_sweep(sid, lambda r, n: pltpu.sync_copy(
        cnt_sh.at[pl.ds(r, n)], cnt_hbm.at[cid, pl.ds(r, n)]))


def _agg_body(src_hbm, dst_hbm, xc_hbm, rc_hbm, gate_hbm,
              agg_hbm,
              sidx0, sidx1, didx0, didx1, xcg0, xcg1, rc0, rc1, gv0, gv1,
              agg_sh, sem0, sem1):
    cid = lax.axis_index("c")
    sid = lax.axis_index("s")
    wid = sid * 2 + cid
    sidx = (sidx0, sidx1)
    didx = (didx0, didx1)
    xcg = (xcg0, xcg1)
    rcv = (rc0, rc1)
    gv = (gv0, gv1)
    sem = (sem0, sem1)

    def fill(i, _):
        for j in range(8):
            rc0[i, 16 * j:16 * (j + 1)] = jnp.zeros((16,), jnp.float32)
        return 0
    lax.fori_loop(0, KG, fill, 0)

    row0 = sid * TILE_ROWS
    for k in range(9):
        pltpu.sync_copy(rc0, agg_sh.at[pl.ds(row0 + 64 * k, 64)])
    pltpu.sync_copy(rc0.at[pl.ds(0, 48)], agg_sh.at[pl.ds(row0 + 576, 48)])

    @pl.when(sid == 0)
    def _():
        pltpu.sync_copy(rc0.at[pl.ds(0, _REM)],
                        agg_sh.at[pl.ds(_REM_OFF, _REM)])
    plsc.subcore_barrier()

    def issue(s, b):
        base = b * KG
        pltpu.sync_copy(src_hbm.at[pl.ds(base, KG)], sidx[s])
        pltpu.sync_copy(dst_hbm.at[pl.ds(base, KG)], didx[s])
        pltpu.async_copy(xc_hbm.at[sidx[s]], xcg[s], sem[s])
        pltpu.async_copy(rc_hbm.at[pl.ds(base, KG)], rcv[s], sem[s])
        pltpu.async_copy(gate_hbm.at[pl.ds(base, KG)], gv[s], sem[s])

    def finish(s, b):
        base = b * KG
        pltpu.make_async_copy(xc_hbm.at[sidx[s]], xcg[s], sem[s]).wait()
        pltpu.make_async_copy(rc_hbm.at[pl.ds(base, KG)], rcv[s],
                              sem[s]).wait()
        pltpu.make_async_copy(gate_hbm.at[pl.ds(base, KG)], gv[s],
                              sem[s]).wait()

        def ebody(ii, _):
            for u in range(4):
                e = 4 * ii + u
                g = gv[s][e, :]
                for j in range(8):
                    sl = slice(16 * j, 16 * (j + 1))
                    rcv[s][e, sl] = g * jnp.maximum(
                        xcg[s][e, sl] + rcv[s][e, sl], 0.0)
            return 0
        lax.fori_loop(0, KG // 4, ebody, 0)

        pltpu.sync_copy(rcv[s], agg_sh.at[didx[s]], add=True)

    issue(0, wid)

    def body(k, _):
        b0 = wid + 32 * (2 * k)
        b1 = b0 + 32
        b2 = b0 + 64

        @pl.when(b1 < NBLK_G)
        def _():
            issue(1, b1)

        @pl.when(b0 < NBLK_G)
        def _():
            finish(0, b0)

        @pl.when(b2 < NBLK_G)
        def _():
            issue(0, b2)

        @pl.when(b1 < NBLK_G)
        def _():
            finish(1, b1)
        return 0
    lax.fori_loop(0, PAIRS_G, body, 0)

    plsc.subcore_barrier()
    _tile_sweep(sid, lambda r, n: pltpu.sync_copy(
        agg_sh.at[pl.ds(r, n)], agg_hbm.at[cid, pl.ds(r, n)]))


_gate_call = pl.kernel(
    _gate_body,
    out_type=[
        jax.ShapeDtypeStruct((E, 16), jnp.float32),
        jax.ShapeDtypeStruct((E, 16), jnp.float32),
    ],
    mesh=_mesh,
    scratch_types=[
        pltpu.VMEM((KW,), jnp.int32),
        pltpu.VMEM((KW,), jnp.int32),
        pltpu.VMEM((KW,), jnp.int32),
        pltpu.VMEM((KW,), jnp.int32),
        pltpu.VMEM((KW, HID), jnp.float32),
        pltpu.VMEM((KW, HID), jnp.float32),
        pltpu.VMEM((KW, HID), jnp.float32),
        pltpu.VMEM((KW, HID), jnp.float32),
        pltpu.VMEM((KW, 16), jnp.float32),
        pltpu.VMEM((KW, 16), jnp.float32),
        pltpu.VMEM((KW, 16), jnp.float32),
        pltpu.VMEM((KW, 16), jnp.float32),
        pltpu.VMEM((2, 128), jnp.float32),
        pltpu.SemaphoreType.DMA,
        pltpu.SemaphoreType.DMA,
    ],
)

_cnt_call = pl.kernel(
    _cnt_body,
    out_type=jax.ShapeDtypeStruct((2, N, 128), jnp.float32),
    mesh=_mesh,
    scratch_types=[
        pltpu.VMEM((KW,), jnp.int32),
        pltpu.VMEM((KW,), jnp.int32),
        pltpu.VMEM((KW, 128), jnp.float32),
        pltpu.VMEM_SHARED((N, 128), jnp.float32),
        pltpu.SemaphoreType.DMA,
        pltpu.SemaphoreType.DMA,
    ],
)

_agg_call = pl.kernel(
    _agg_body,
    out_type=jax.ShapeDtypeStruct((2, N, HID), jnp.float32),
    mesh=_mesh,
    scratch_types=[
        pltpu.VMEM((KG,), jnp.int32),
        pltpu.VMEM((KG,), jnp.int32),
        pltpu.VMEM((KG,), jnp.int32),
        pltpu.VMEM((KG,), jnp.int32),
        pltpu.VMEM((KG, HID), jnp.float32),
        pltpu.VMEM((KG, HID), jnp.float32),
        pltpu.VMEM((KG, HID), jnp.float32),
        pltpu.VMEM((KG, HID), jnp.float32),
        pltpu.VMEM((KG, 16), jnp.float32),
        pltpu.VMEM((KG, 16), jnp.float32),
        pltpu.VMEM_SHARED((N, HID), jnp.float32),
        pltpu.SemaphoreType.DMA,
        pltpu.SemaphoreType.DMA,
    ],
)


def _proj_in(ee, pe32, wt, wb, bi, wmt, a0, a1, b0, b1, g0, g1):
    nspec = pl.BlockSpec((BLK_N, EMB), lambda i: (i, 0))
    w3264 = pl.BlockSpec((32, 64), lambda i: (0, 0))
    b64 = pl.BlockSpec((1, 64), lambda i: (0, 0))
    return pl.pallas_call(
        _proj_in_kernel,
        grid=(N // BLK_N,),
        in_specs=[
            nspec,
            pl.BlockSpec((BLK_N, 32), lambda i: (i, 0)),
            pl.BlockSpec((EMB, HID), lambda i: (0, 0)),
            pl.BlockSpec((32, HID), lambda i: (0, 0)),
            pl.BlockSpec((1, HID), lambda i: (0, 0)),
            pl.BlockSpec((HID, HID), lambda i: (0, 0)),
            w3264, w3264, w3264, w3264, b64, b64,
        ],
        out_specs=[nspec, nspec, nspec, nspec],
        out_shape=[jax.ShapeDtypeStruct((N, HID), jnp.float32)] * 4,
    )(ee, pe32, wt, wb, bi, wmt, a0, a1, b0, b1, g0, g1)


def _rc(rel, wr, br, wm, bm):
    return pl.pallas_call(
        _rc_kernel,
        grid=(E // BLK_E,),
        in_specs=[
            pl.BlockSpec((BLK_E, EMB), lambda i: (i, 0)),
            pl.BlockSpec((EMB, REL), lambda i: (0, 0)),
            pl.BlockSpec((1, REL), lambda i: (0, 0)),
            pl.BlockSpec((REL, HID), lambda i: (0, 0)),
            pl.BlockSpec((1, HID), lambda i: (0, 0)),
        ],
        out_specs=pl.BlockSpec((BLK_E, HID), lambda i: (i, 0)),
        out_shape=jax.ShapeDtypeStruct((E, HID), jnp.float32),
    )(rel, wr, br, wm, bm)


def _upd(x, aggp, cnt16, wt, wb, b, wn, bn, final):
    nspec = pl.BlockSpec((BLK_N, HID), lambda i: (i, 0))
    wspec = pl.BlockSpec((HID, HID), lambda i: (0, 0))
    bspec = pl.BlockSpec((1, HID), lambda i: (0, 0))
    return pl.pallas_call(
        functools.partial(_upd_kernel, final=final),
        grid=(N // BLK_N,),
        in_specs=[
            nspec,
            pl.BlockSpec((2, BLK_N, HID), lambda i: (0, i, 0)),
            pl.BlockSpec((2, BLK_N, 128), lambda i: (0, i, 0)),
            wspec, wspec, bspec, wspec, bspec,
        ],
        out_specs=[nspec, nspec],
        out_shape=[jax.ShapeDtypeStruct((N, HID), jnp.float32)] * 2,
    )(x, aggp, cnt16, wt, wb, b, wn, bn)


def kernel(entity_embs, pe, edge_index, relation_embs_per_edge, W_in, b_in,
           W_rel, b_rel, W_msg, b_msg, W_g1, b_g1, W_g2, b_g2, W_upd, b_upd,
           W_out, b_out):
    src = edge_index[0]
    dst = edge_index[1]
    pe32 = jnp.pad(pe, ((0, 0), (0, 32 - PE)))

    w2e = jnp.zeros((L, 128), jnp.float32)
    w2e = w2e.at[:, :64].set(W_g2[:, :, 0])
    w2e = w2e.at[:, 64:80].set(jnp.broadcast_to(b_g2.reshape(L, 1), (L, 16)))

    x, xc, s_tab, d_tab = _proj_in(
        entity_embs, pe32, W_in[:EMB],
        jnp.pad(W_in[EMB:], ((0, 32 - PE), (0, 0))),
        b_in.reshape(1, HID), W_msg[0][:HID],
        jnp.pad(W_g1[0][:PE], ((0, 1), (0, 0))),
        jnp.pad(W_g1[1][:PE], ((0, 1), (0, 0))),
        jnp.pad(W_g1[0][PE:], ((0, 1), (0, 0))),
        jnp.pad(W_g1[1][PE:], ((0, 1), (0, 0))),
        b_g1[0].reshape(1, 64), b_g1[1].reshape(1, 64))

    gate0, gate1 = _gate_call(src, dst, s_tab, d_tab, w2e)
    cnt16 = _cnt_call(dst)

    rc0 = _rc(relation_embs_per_edge, W_rel, b_rel.reshape(1, REL),
              W_msg[0][HID:], b_msg[0].reshape(1, HID))
    rc1 = _rc(relation_embs_per_edge, W_rel, b_rel.reshape(1, REL),
              W_msg[1][HID:], b_msg[1].reshape(1, HID))

    agg0p = _agg_call(src, dst, xc, rc0, gate0)
    x, xc = _upd(x, agg0p, cnt16, W_upd[0][:HID], W_upd[0][HID:],
                 b_upd[0].reshape(1, HID), W_msg[1][:HID],
                 jnp.zeros((1, HID), jnp.float32), final=False)

    agg1p = _agg_call(src, dst, xc, rc1, gate1)
    _, out = _upd(x, agg1p, cnt16, W_upd[1][:HID], W_upd[1][HID:],
                  b_upd[1].reshape(1, HID), W_out,
                  b_out.reshape(1, EMB), final=True)
    return out


# speedup vs baseline: 4.5950x; 1.1012x over previous
import functools

import jax
import jax.numpy as jnp
from jax import lax
from jax.experimental import pallas as pl
from jax.experimental.pallas import tpu as pltpu
from jax.experimental.pallas import tpu_sc as plsc

N = 10000
E = 320000
EMB = 128
HID = 128
REL = 64
PE = 31
L = 2

BLK_E = 512
BLK_N = 1000
TILE_ROWS = 624
_CHUNKS = ((0, 128), (128, 128), (256, 128), (384, 128), (512, 112))
_REM_OFF = 16 * TILE_ROWS
_REM = N - _REM_OFF

_mesh = plsc.VectorSubcoreMesh(core_axis_name="c", subcore_axis_name="s")


def _proj_in_kernel(ee_ref, pe_ref, wt_ref, wb_ref, bi_ref, wmt_ref,
                    a0_ref, a1_ref, b0_ref, b1_ref, g0_ref, g1_ref,
                    x_ref, xc_ref, s_ref, d_ref):
    pe = pe_ref[...]
    x = jnp.dot(ee_ref[...], wt_ref[...], preferred_element_type=jnp.float32)
    x = x + jnp.dot(pe, wb_ref[...], preferred_element_type=jnp.float32)
    x = jnp.maximum(x + bi_ref[...], 0.0)
    x_ref[...] = x
    xc_ref[...] = jnp.dot(x, wmt_ref[...], preferred_element_type=jnp.float32)
    s0 = jnp.dot(pe, a0_ref[...], preferred_element_type=jnp.float32)
    s1 = jnp.dot(pe, a1_ref[...], preferred_element_type=jnp.float32)
    s_ref[...] = jnp.concatenate([s0, s1], axis=1)
    d0 = jnp.dot(pe, b0_ref[...], preferred_element_type=jnp.float32)
    d1 = jnp.dot(pe, b1_ref[...], preferred_element_type=jnp.float32)
    d_ref[...] = jnp.concatenate([d0 + g0_ref[...], d1 + g1_ref[...]], axis=1)


def _rc_kernel(rel_ref, wr_ref, br_ref, wm0_ref, bm0_ref, wm1_ref, bm1_ref,
               rc0_ref, rc1_ref):
    ea = jnp.dot(rel_ref[...], wr_ref[...], preferred_element_type=jnp.float32)
    ea = ea + br_ref[...]
    rc0 = jnp.dot(ea, wm0_ref[...], preferred_element_type=jnp.float32)
    rc0_ref[...] = rc0 + bm0_ref[...]
    rc1 = jnp.dot(ea, wm1_ref[...], preferred_element_type=jnp.float32)
    rc1_ref[...] = rc1 + bm1_ref[...]


def _upd_kernel(x_ref, aggp_ref, cnt_ref, wt_ref, wb_ref, b_ref, wn_ref,
                bn_ref, x2_ref, xc2_ref, final):
    aggs = aggp_ref[0] + aggp_ref[1]
    cnt = jnp.maximum(cnt_ref[0, :, :1] + cnt_ref[1, :, :1], 1.0)
    agg = aggs / cnt
    x = jnp.dot(x_ref[...], wt_ref[...], preferred_element_type=jnp.float32)
    x = x + jnp.dot(agg, wb_ref[...], preferred_element_type=jnp.float32)
    x = jnp.maximum(x + b_ref[...], 0.0)
    x2_ref[...] = x
    y = jnp.dot(x, wn_ref[...], preferred_element_type=jnp.float32)
    if final:
        xc2_ref[...] = y + bn_ref[...]
    else:
        xc2_ref[...] = y


_GDN = lax.GatherDimensionNumbers(
    offset_dims=(), collapsed_slice_dims=(0,), start_index_map=(0,))


def _allsum(v):
    for s in (8, 4, 2, 1):
        idx = lax.iota(jnp.int32, 16) ^ s
        v = v + lax.gather(v, idx[:, None], _GDN, (1,),
                           mode=lax.GatherScatterMode.PROMISE_IN_BOUNDS)
    return v


def _tile_sweep(sid, copy_fn):
    row0 = sid * TILE_ROWS
    for off, nr in _CHUNKS:
        copy_fn(row0 + off, nr)

    @pl.when(sid == 0)
    def _():
        copy_fn(_REM_OFF, _REM)


KG = 64
NBLK_G = E // KG
MAX_WIN_G = (NBLK_G + 31) // 32
PAIRS_G = (MAX_WIN_G + 1) // 2
KW = 80
NBLK_W = E // KW
MAX_WIN_W = (NBLK_W + 31) // 32
PAIRS_W = (MAX_WIN_W + 1) // 2


def _gate_from_rows(sg_v, dg_v, e, w2v, bg2v, l):
    acc = None
    for j in range(4):
        c = 64 * l + 16 * j
        h = jnp.maximum(sg_v[e, c:c + 16] + dg_v[e, c:c + 16], 0.0)
        t = h * w2v[4 * l + j]
        acc = t if acc is None else acc + t
    logit = _allsum(acc) + bg2v[l]
    return 1.0 / (1.0 + jnp.exp(-logit))


def _gate_body(src_hbm, dst_hbm, s_hbm, d_hbm, w2e_hbm,
                   gate0_hbm, gate1_hbm,
                   sidx0, sidx1, didx0, didx1, sg0, sg1, dg0, dg1,
                   g00, g01, g10, g11, w2_v, sem0, sem1):
    cid = lax.axis_index("c")
    sid = lax.axis_index("s")
    wid = sid * 2 + cid
    sidx = (sidx0, sidx1)
    didx = (didx0, didx1)
    sg = (sg0, sg1)
    dg = (dg0, dg1)
    g0_v = (g00, g01)
    g1_v = (g10, g11)
    sem = (sem0, sem1)

    pltpu.sync_copy(w2e_hbm, w2_v)

    w2v = [w2_v[j // 4, 16 * (j % 4):16 * (j % 4) + 16] for j in range(8)]
    bg2v = [w2_v[l, 64:80] for l in range(L)]

    def issue(s, b):
        base = b * KW
        pltpu.sync_copy(src_hbm.at[pl.ds(base, KW)], sidx[s])
        pltpu.sync_copy(dst_hbm.at[pl.ds(base, KW)], didx[s])
        pltpu.async_copy(s_hbm.at[sidx[s]], sg[s], sem[s])
        pltpu.async_copy(d_hbm.at[didx[s]], dg[s], sem[s])

    def finish(s, b):
        base = b * KW
        pltpu.make_async_copy(s_hbm.at[sidx[s]], sg[s], sem[s]).wait()
        pltpu.make_async_copy(d_hbm.at[didx[s]], dg[s], sem[s]).wait()

        def ebody(ii, _):
            for u in range(2):
                e = 2 * ii + u
                g0_v[s][e, :] = _gate_from_rows(sg[s], dg[s], e, w2v, bg2v, 0)
                g1_v[s][e, :] = _gate_from_rows(sg[s], dg[s], e, w2v, bg2v, 1)
            return 0
        lax.fori_loop(0, KW // 2, ebody, 0)

        pltpu.sync_copy(g0_v[s], gate0_hbm.at[pl.ds(base, KW)])
        pltpu.sync_copy(g1_v[s], gate1_hbm.at[pl.ds(base, KW)])

    issue(0, wid)

    def body(k, _):
        b0 = wid + 32 * (2 * k)
        b1 = b0 + 32
        b2 = b0 + 64

        @pl.when(b1 < NBLK_W)
        def _():
            issue(1, b1)

        @pl.when(b0 < NBLK_W)
        def _():
            finish(0, b0)

        @pl.when(b2 < NBLK_W)
        def _():
            issue(0, b2)

        @pl.when(b1 < NBLK_W)
        def _():
            finish(1, b1)
        return 0
    lax.fori_loop(0, PAIRS_W, body, 0)


def _cnt_body(dst_hbm, cnt_hbm,
              didx0, didx1, ones_v, cnt_sh, sem0, sem1):
    cid = lax.axis_index("c")
    sid = lax.axis_index("s")
    wid = sid * 2 + cid
    didx = (didx0, didx1)
    sem = (sem0, sem1)

    def fill(i, _):
        for j in range(8):
            ones_v[i, 16 * j:16 * (j + 1)] = jnp.zeros((16,), jnp.float32)
        return 0
    lax.fori_loop(0, KW, fill, 0)

    row0 = sid * TILE_ROWS
    for k in range(7):
        pltpu.sync_copy(ones_v, cnt_sh.at[pl.ds(row0 + 80 * k, 80)])
    pltpu.sync_copy(ones_v.at[pl.ds(0, 64)],
                    cnt_sh.at[pl.ds(row0 + 560, 64)])

    @pl.when(sid == 0)
    def _():
        pltpu.sync_copy(ones_v.at[pl.ds(0, _REM)],
                        cnt_sh.at[pl.ds(_REM_OFF, _REM)])

    def refill(i, _):
        for j in range(8):
            ones_v[i, 16 * j:16 * (j + 1)] = jnp.ones((16,), jnp.float32)
        return 0
    lax.fori_loop(0, KW, refill, 0)
    plsc.subcore_barrier()

    def issue(s, b):
        pltpu.async_copy(dst_hbm.at[pl.ds(b * KW, KW)], didx[s], sem[s])

    def finish(s, b):
        pltpu.make_async_copy(dst_hbm.at[pl.ds(b * KW, KW)], didx[s],
                              sem[s]).wait()
        pltpu.sync_copy(ones_v, cnt_sh.at[didx[s]], add=True)

    issue(0, wid)

    def body(k, _):
        b0 = wid + 32 * (2 * k)
        b1 = b0 + 32
        b2 = b0 + 64

        @pl.when(b1 < NBLK_W)
        def _():
            issue(1, b1)

        @pl.when(b0 < NBLK_W)
        def _():
            finish(0, b0)

        @pl.when(b2 < NBLK_W)
        def _():
            issue(0, b2)

        @pl.when(b1 < NBLK_W)
        def _():
            finish(1, b1)
        return 0
    lax.fori_loop(0, PAIRS_W, body, 0)

    plsc.subcore_barrier()
    _tile_sweep(sid, lambda r, n: pltpu.sync_copy(
        cnt_sh.at[pl.ds(r, n)], cnt_hbm.at[cid, pl.ds(r, n)]))


def _agg_body(src_hbm, dst_hbm, xc_hbm, rc_hbm, gate_hbm,
              agg_hbm,
              sidx0, sidx1, didx0, didx1, xcg0, xcg1, rc0, rc1, gv0, gv1,
              agg_sh, sem0, sem1):
    cid = lax.axis_index("c")
    sid = lax.axis_index("s")
    wid = sid * 2 + cid
    sidx = (sidx0, sidx1)
    didx = (didx0, didx1)
    xcg = (xcg0, xcg1)
    rcv = (rc0, rc1)
    gv = (gv0, gv1)
    sem = (sem0, sem1)

    def fill(i, _):
        for j in range(8):
            rc0[i, 16 * j:16 * (j + 1)] = jnp.zeros((16,), jnp.float32)
        return 0
    lax.fori_loop(0, KG, fill, 0)

    row0 = sid * TILE_ROWS
    for k in range(9):
        pltpu.sync_copy(rc0, agg_sh.at[pl.ds(row0 + 64 * k, 64)])
    pltpu.sync_copy(rc0.at[pl.ds(0, 48)], agg_sh.at[pl.ds(row0 + 576, 48)])

    @pl.when(sid == 0)
    def _():
        pltpu.sync_copy(rc0.at[pl.ds(0, _REM)],
                        agg_sh.at[pl.ds(_REM_OFF, _REM)])
    plsc.subcore_barrier()

    def issue(s, b):
        base = b * KG
        pltpu.sync_copy(src_hbm.at[pl.ds(base, KG)], sidx[s])
        pltpu.sync_copy(dst_hbm.at[pl.ds(base, KG)], didx[s])
        pltpu.async_copy(xc_hbm.at[sidx[s]], xcg[s], sem[s])
        pltpu.async_copy(rc_hbm.at[pl.ds(base, KG)], rcv[s], sem[s])
        pltpu.async_copy(gate_hbm.at[pl.ds(base, KG)], gv[s], sem[s])

    def finish(s, b):
        base = b * KG
        pltpu.make_async_copy(xc_hbm.at[sidx[s]], xcg[s], sem[s]).wait()
        pltpu.make_async_copy(rc_hbm.at[pl.ds(base, KG)], rcv[s],
                              sem[s]).wait()
        pltpu.make_async_copy(gate_hbm.at[pl.ds(base, KG)], gv[s],
                              sem[s]).wait()

        def ebody(ii, _):
            for u in range(4):
                e = 4 * ii + u
                g = gv[s][e, :]
                for j in range(8):
                    sl = slice(16 * j, 16 * (j + 1))
                    rcv[s][e, sl] = g * jnp.maximum(
                        xcg[s][e, sl] + rcv[s][e, sl], 0.0)
            return 0
        lax.fori_loop(0, KG // 4, ebody, 0)

        pltpu.sync_copy(rcv[s], agg_sh.at[didx[s]], add=True)

    issue(0, wid)

    def body(k, _):
        b0 = wid + 32 * (2 * k)
        b1 = b0 + 32
        b2 = b0 + 64

        @pl.when(b1 < NBLK_G)
        def _():
            issue(1, b1)

        @pl.when(b0 < NBLK_G)
        def _():
            finish(0, b0)

        @pl.when(b2 < NBLK_G)
        def _():
            issue(0, b2)

        @pl.when(b1 < NBLK_G)
        def _():
            finish(1, b1)
        return 0
    lax.fori_loop(0, PAIRS_G, body, 0)

    plsc.subcore_barrier()
    _tile_sweep(sid, lambda r, n: pltpu.sync_copy(
        agg_sh.at[pl.ds(r, n)], agg_hbm.at[cid, pl.ds(r, n)]))


_gate_call = pl.kernel(
    _gate_body,
    out_type=[
        jax.ShapeDtypeStruct((E, 16), jnp.float32),
        jax.ShapeDtypeStruct((E, 16), jnp.float32),
    ],
    mesh=_mesh,
    scratch_types=[
        pltpu.VMEM((KW,), jnp.int32),
        pltpu.VMEM((KW,), jnp.int32),
        pltpu.VMEM((KW,), jnp.int32),
        pltpu.VMEM((KW,), jnp.int32),
        pltpu.VMEM((KW, HID), jnp.float32),
        pltpu.VMEM((KW, HID), jnp.float32),
        pltpu.VMEM((KW, HID), jnp.float32),
        pltpu.VMEM((KW, HID), jnp.float32),
        pltpu.VMEM((KW, 16), jnp.float32),
        pltpu.VMEM((KW, 16), jnp.float32),
        pltpu.VMEM((KW, 16), jnp.float32),
        pltpu.VMEM((KW, 16), jnp.float32),
        pltpu.VMEM((2, 128), jnp.float32),
        pltpu.SemaphoreType.DMA,
        pltpu.SemaphoreType.DMA,
    ],
)

_cnt_call = pl.kernel(
    _cnt_body,
    out_type=jax.ShapeDtypeStruct((2, N, 128), jnp.float32),
    mesh=_mesh,
    scratch_types=[
        pltpu.VMEM((KW,), jnp.int32),
        pltpu.VMEM((KW,), jnp.int32),
        pltpu.VMEM((KW, 128), jnp.float32),
        pltpu.VMEM_SHARED((N, 128), jnp.float32),
        pltpu.SemaphoreType.DMA,
        pltpu.SemaphoreType.DMA,
    ],
)

_agg_call = pl.kernel(
    _agg_body,
    out_type=jax.ShapeDtypeStruct((2, N, HID), jnp.float32),
    mesh=_mesh,
    scratch_types=[
        pltpu.VMEM((KG,), jnp.int32),
        pltpu.VMEM((KG,), jnp.int32),
        pltpu.VMEM((KG,), jnp.int32),
        pltpu.VMEM((KG,), jnp.int32),
        pltpu.VMEM((KG, HID), jnp.float32),
        pltpu.VMEM((KG, HID), jnp.float32),
        pltpu.VMEM((KG, HID), jnp.float32),
        pltpu.VMEM((KG, HID), jnp.float32),
        pltpu.VMEM((KG, 16), jnp.float32),
        pltpu.VMEM((KG, 16), jnp.float32),
        pltpu.VMEM_SHARED((N, HID), jnp.float32),
        pltpu.SemaphoreType.DMA,
        pltpu.SemaphoreType.DMA,
    ],
)


def _proj_in(ee, pe32, wt, wb, bi, wmt, a0, a1, b0, b1, g0, g1):
    nspec = pl.BlockSpec((BLK_N, EMB), lambda i: (i, 0))
    w3264 = pl.BlockSpec((32, 64), lambda i: (0, 0))
    b64 = pl.BlockSpec((1, 64), lambda i: (0, 0))
    return pl.pallas_call(
        _proj_in_kernel,
        grid=(N // BLK_N,),
        in_specs=[
            nspec,
            pl.BlockSpec((BLK_N, 32), lambda i: (i, 0)),
            pl.BlockSpec((EMB, HID), lambda i: (0, 0)),
            pl.BlockSpec((32, HID), lambda i: (0, 0)),
            pl.BlockSpec((1, HID), lambda i: (0, 0)),
            pl.BlockSpec((HID, HID), lambda i: (0, 0)),
            w3264, w3264, w3264, w3264, b64, b64,
        ],
        out_specs=[nspec, nspec, nspec, nspec],
        out_shape=[jax.ShapeDtypeStruct((N, HID), jnp.float32)] * 4,
    )(ee, pe32, wt, wb, bi, wmt, a0, a1, b0, b1, g0, g1)


def _rc(rel, wr, br, wm0, bm0, wm1, bm1):
    espec = pl.BlockSpec((BLK_E, HID), lambda i: (i, 0))
    return pl.pallas_call(
        _rc_kernel,
        grid=(E // BLK_E,),
        in_specs=[
            pl.BlockSpec((BLK_E, EMB), lambda i: (i, 0)),
            pl.BlockSpec((EMB, REL), lambda i: (0, 0)),
            pl.BlockSpec((1, REL), lambda i: (0, 0)),
            pl.BlockSpec((REL, HID), lambda i: (0, 0)),
            pl.BlockSpec((1, HID), lambda i: (0, 0)),
            pl.BlockSpec((REL, HID), lambda i: (0, 0)),
            pl.BlockSpec((1, HID), lambda i: (0, 0)),
        ],
        out_specs=[espec, espec],
        out_shape=[jax.ShapeDtypeStruct((E, HID), jnp.float32)] * 2,
    )(rel, wr, br, wm0, bm0, wm1, bm1)


def _upd(x, aggp, cnt16, wt, wb, b, wn, bn, final):
    nspec = pl.BlockSpec((BLK_N, HID), lambda i: (i, 0))
    wspec = pl.BlockSpec((HID, HID), lambda i: (0, 0))
    bspec = pl.BlockSpec((1, HID), lambda i: (0, 0))
    return pl.pallas_call(
        functools.partial(_upd_kernel, final=final),
        grid=(N // BLK_N,),
        in_specs=[
            nspec,
            pl.BlockSpec((2, BLK_N, HID), lambda i: (0, i, 0)),
            pl.BlockSpec((2, BLK_N, 128), lambda i: (0, i, 0)),
            wspec, wspec, bspec, wspec, bspec,
        ],
        out_specs=[nspec, nspec],
        out_shape=[jax.ShapeDtypeStruct((N, HID), jnp.float32)] * 2,
    )(x, aggp, cnt16, wt, wb, b, wn, bn)


def kernel(entity_embs, pe, edge_index, relation_embs_per_edge, W_in, b_in,
           W_rel, b_rel, W_msg, b_msg, W_g1, b_g1, W_g2, b_g2, W_upd, b_upd,
           W_out, b_out):
    src = edge_index[0]
    dst = edge_index[1]
    pe32 = jnp.pad(pe, ((0, 0), (0, 32 - PE)))

    w2e = jnp.zeros((L, 128), jnp.float32)
    w2e = w2e.at[:, :64].set(W_g2[:, :, 0])
    w2e = w2e.at[:, 64:80].set(jnp.broadcast_to(b_g2.reshape(L, 1), (L, 16)))

    x, xc, s_tab, d_tab = _proj_in(
        entity_embs, pe32, W_in[:EMB],
        jnp.pad(W_in[EMB:], ((0, 32 - PE), (0, 0))),
        b_in.reshape(1, HID), W_msg[0][:HID],
        jnp.pad(W_g1[0][:PE], ((0, 1), (0, 0))),
        jnp.pad(W_g1[1][:PE], ((0, 1), (0, 0))),
        jnp.pad(W_g1[0][PE:], ((0, 1), (0, 0))),
        jnp.pad(W_g1[1][PE:], ((0, 1), (0, 0))),
        b_g1[0].reshape(1, 64), b_g1[1].reshape(1, 64))

    gate0, gate1 = _gate_call(src, dst, s_tab, d_tab, w2e)
    cnt16 = _cnt_call(dst)

    rc0, rc1 = _rc(relation_embs_per_edge, W_rel, b_rel.reshape(1, REL),
                   W_msg[0][HID:], b_msg[0].reshape(1, HID),
                   W_msg[1][HID:], b_msg[1].reshape(1, HID))

    agg0p = _agg_call(src, dst, xc, rc0, gate0)
    x, xc = _upd(x, agg0p, cnt16, W_upd[0][:HID], W_upd[0][HID:],
                 b_upd[0].reshape(1, HID), W_msg[1][:HID],
                 jnp.zeros((1, HID), jnp.float32), final=False)

    agg1p = _agg_call(src, dst, xc, rc1, gate1)
    _, out = _upd(x, agg1p, cnt16, W_upd[1][:HID], W_upd[1][HID:],
                  b_upd[1].reshape(1, HID), W_out,
                  b_out.reshape(1, EMB), final=True)
    return out
